# baseline (speedup 1.0000x reference)
import jax
import jax.numpy as jnp
from jax.experimental import pallas as pl

OFFSET = 0.001
BLOCK_ROWS = 4096
SPLIT = 2


def _body(x_ref, o_ref):
    o_ref[...] = 1.0 / (jnp.abs(x_ref[...]) + OFFSET)


def kernel(xyz):
    n, d = xyz.shape
    x3 = xyz.reshape(SPLIT, n // SPLIT, d)
    out = pl.pallas_call(
        _body,
        grid=(n // SPLIT // BLOCK_ROWS,),
        in_specs=[pl.BlockSpec((SPLIT, BLOCK_ROWS, d), lambda i: (0, i, 0))],
        out_specs=pl.BlockSpec((SPLIT, BLOCK_ROWS, d), lambda i: (0, i, 0)),
        out_shape=jax.ShapeDtypeStruct(x3.shape, xyz.dtype),
    )(x3)
    return out.reshape(n, d)
